# Initial kernel scaffold; baseline (speedup 1.0000x reference)
import jax
import jax.numpy as jnp
from jax.experimental import pallas as pl


def kernel(x, edge_index, W1, b1, W2, b2, Wd, bd):
    raise NotImplementedError("write your pallas kernel here")


# speedup vs baseline: 16.5895x; 16.5895x over previous
import functools

import jax
import jax.numpy as jnp
from jax import lax
from jax.experimental import pallas as pl
from jax.experimental.pallas import tpu as pltpu
from jax.experimental.pallas import tpu_sc as plsc

_N = 10000
_E = 320000
_F = 128
_H = 64
_C = 4

_CH = 128
_CPT = 79
_NT = 32
_R = _CPT * _CH
_EPAD = _NT * _CPT * _CH
_BR = 128
_NB = _R // _BR

_mesh = plsc.VectorSubcoreMesh(core_axis_name="c", subcore_axis_name="s")


@functools.partial(
    pl.kernel,
    out_type=jax.ShapeDtypeStruct((2, _R, 16), jnp.float32),
    mesh=_mesh,
    scratch_types=[
        pltpu.VMEM((_CPT, _CH), jnp.int32),
        pltpu.VMEM((_CH, 16), jnp.float32),
        pltpu.VMEM_SHARED((_R, 16), jnp.float32),
    ],
)
def _sc_degree(dstp_hbm, ones_hbm, zeros_hbm, out_hbm, dst_v, ones_v, acc_sh):
    c = lax.axis_index("c")
    s = lax.axis_index("s")
    gid = c * 16 + s
    pltpu.sync_copy(dstp_hbm.at[gid], dst_v)
    pltpu.sync_copy(ones_hbm, ones_v)

    @pl.when(s == 0)
    def _():
        pltpu.sync_copy(zeros_hbm, acc_sh)

    plsc.subcore_barrier()

    def step(j, carry):
        pltpu.sync_copy(ones_v, acc_sh.at[dst_v.at[j]], add=True)
        return carry

    lax.fori_loop(0, _CPT, step, 0)
    plsc.subcore_barrier()

    @pl.when(s == 0)
    def _():
        pltpu.sync_copy(acc_sh, out_hbm.at[c])


@functools.partial(
    pl.kernel,
    out_type=jax.ShapeDtypeStruct((2, _R, _H), jnp.float32),
    mesh=_mesh,
    compiler_params=pltpu.CompilerParams(use_tc_tiling_on_sc=False),
    scratch_types=[
        pltpu.VMEM((_CPT, _CH), jnp.int32),
        pltpu.VMEM((_CPT, _CH), jnp.int32),
        pltpu.VMEM((_CH, _H), jnp.float32),
        pltpu.VMEM_SHARED((_R, _H), jnp.float32),
        pltpu.SemaphoreType.DMA,
    ],
)
def _sc_scatter(table_hbm, srcp_hbm, dstp_hbm, zeros_hbm, out_hbm,
                src_v, dst_v, rows_v, acc_sh, sem):
    c = lax.axis_index("c")
    s = lax.axis_index("s")
    gid = c * 16 + s
    pltpu.sync_copy(srcp_hbm.at[gid], src_v)
    pltpu.sync_copy(dstp_hbm.at[gid], dst_v)

    @pl.when(s == 0)
    def _():
        pltpu.sync_copy(zeros_hbm, acc_sh)

    plsc.subcore_barrier()

    def step(j, carry):
        pltpu.async_copy(table_hbm.at[src_v.at[j]], rows_v, sem).wait()
        pltpu.sync_copy(rows_v, acc_sh.at[dst_v.at[j]], add=True)
        return carry

    lax.fori_loop(0, _CPT, step, 0)
    plsc.subcore_barrier()

    @pl.when(s == 0)
    def _():
        pltpu.sync_copy(acc_sh, out_hbm.at[c])


def _norm_col(degp, j):
    deg = degp[0] + degp[1]
    degc = deg[:, 0:1] + 1.0
    rid = j * _BR + lax.broadcasted_iota(jnp.int32, (_BR, 1), 0)
    return jnp.where(rid < _N, lax.rsqrt(degc), 0.0)


def _tc_layer1_body(x_ref, degp_ref, w_ref, b_ref, g_ref):
    norm = _norm_col(degp_ref, pl.program_id(0))
    xw = jnp.dot(x_ref[...], w_ref[...], preferred_element_type=jnp.float32)
    g_ref[...] = (xw + b_ref[...]) * norm


def _tc_layer1(x_pad, degp, W1, b1):
    return pl.pallas_call(
        _tc_layer1_body,
        grid=(_NB,),
        in_specs=[
            pl.BlockSpec((_BR, _F), lambda j: (j, 0)),
            pl.BlockSpec((2, _BR, 16), lambda j: (0, j, 0)),
            pl.BlockSpec((_F, _H), lambda j: (0, 0)),
            pl.BlockSpec((1, _H), lambda j: (0, 0)),
        ],
        out_specs=pl.BlockSpec((_BR, _H), lambda j: (j, 0)),
        out_shape=jax.ShapeDtypeStruct((_R, _H), jnp.float32),
    )(x_pad, degp, W1, b1)


def _tc_layer2_body(acc_ref, g1_ref, degp_ref, w_ref, b_ref, g2_ref):
    norm = _norm_col(degp_ref, pl.program_id(0))
    h1 = jnp.maximum((acc_ref[0] + acc_ref[1] + g1_ref[...]) * norm, 0.0)
    hw = jnp.dot(h1, w_ref[...], preferred_element_type=jnp.float32)
    g2_ref[...] = (hw + b_ref[...]) * norm


def _tc_layer2(acc1, g1, degp, W2, b2):
    return pl.pallas_call(
        _tc_layer2_body,
        grid=(_NB,),
        in_specs=[
            pl.BlockSpec((2, _BR, _H), lambda j: (0, j, 0)),
            pl.BlockSpec((_BR, _H), lambda j: (j, 0)),
            pl.BlockSpec((2, _BR, 16), lambda j: (0, j, 0)),
            pl.BlockSpec((_H, _H), lambda j: (0, 0)),
            pl.BlockSpec((1, _H), lambda j: (0, 0)),
        ],
        out_specs=pl.BlockSpec((_BR, _H), lambda j: (j, 0)),
        out_shape=jax.ShapeDtypeStruct((_R, _H), jnp.float32),
    )(acc1, g1, degp, W2, b2)


def _tc_head_body(acc_ref, g2_ref, degp_ref, wd_ref, bd_ref, out_ref, psum):
    j = pl.program_id(0)
    norm = _norm_col(degp_ref, j)
    h2 = jnp.maximum((acc_ref[0] + acc_ref[1] + g2_ref[...]) * norm, 0.0)

    @pl.when(j == 0)
    def _():
        psum[...] = jnp.zeros_like(psum)

    psum[...] += jnp.sum(h2, axis=0, keepdims=True)

    @pl.when(j == _NB - 1)
    def _():
        logits = jnp.dot(psum[...], wd_ref[...],
                         preferred_element_type=jnp.float32) + bd_ref[...]
        m = jnp.max(logits, axis=-1, keepdims=True)
        e = jnp.exp(logits - m)
        out_ref[...] = e / jnp.sum(e, axis=-1, keepdims=True)


def _tc_head(acc2, g2, degp, Wd_pad, bd_pad):
    return pl.pallas_call(
        _tc_head_body,
        grid=(_NB,),
        in_specs=[
            pl.BlockSpec((2, _BR, _H), lambda j: (0, j, 0)),
            pl.BlockSpec((_BR, _H), lambda j: (j, 0)),
            pl.BlockSpec((2, _BR, 16), lambda j: (0, j, 0)),
            pl.BlockSpec((_H, 128), lambda j: (0, 0)),
            pl.BlockSpec((1, 128), lambda j: (0, 0)),
        ],
        out_specs=pl.BlockSpec((1, 128), lambda j: (0, 0)),
        out_shape=jax.ShapeDtypeStruct((1, 128), jnp.float32),
        scratch_shapes=[pltpu.VMEM((1, _H), jnp.float32)],
    )(acc2, g2, degp, Wd_pad, bd_pad)


def kernel(x, edge_index, W1, b1, W2, b2, Wd, bd):
    src = edge_index[0].astype(jnp.int32)
    dst = edge_index[1].astype(jnp.int32)
    pad_e = _EPAD - _E
    srcp = jnp.concatenate(
        [src, jnp.zeros((pad_e,), jnp.int32)]).reshape(_NT, _CPT, _CH)
    dstp = jnp.concatenate(
        [dst, jnp.full((pad_e,), _R - 1, jnp.int32)]).reshape(_NT, _CPT, _CH)
    x_pad = jnp.pad(x, ((0, _R - _N), (0, 0)))
    zeros64 = jnp.zeros((_R, _H), jnp.float32)
    zeros16 = jnp.zeros((_R, 16), jnp.float32)
    ones16 = jnp.ones((_CH, 16), jnp.float32)
    b1r = b1.reshape(1, _H)
    b2r = b2.reshape(1, _H)
    Wd_pad = jnp.zeros((_H, 128), jnp.float32).at[:, :_C].set(Wd)
    bd_pad = jnp.full((1, 128), -1e30, jnp.float32).at[0, :_C].set(bd)

    degp = _sc_degree(dstp, ones16, zeros16)
    g1 = _tc_layer1(x_pad, degp, W1, b1r)
    acc1 = _sc_scatter(g1, srcp, dstp, zeros64)
    g2 = _tc_layer2(acc1, g1, degp, W2, b2r)
    acc2 = _sc_scatter(g2, srcp, dstp, zeros64)
    out = _tc_head(acc2, g2, degp, Wd_pad, bd_pad)
    return out[:, :_C]
